# baseline (speedup 1.0000x reference)
import functools

import jax
import jax.numpy as jnp
from jax import lax
from jax.experimental import pallas as pl
from jax.experimental.pallas import tpu as pltpu
from jax.experimental.pallas import tpu_sc as plsc

_L = 16


def kernel(input, input_position, input_table, pos_table):
    B = input.shape[0]
    D = input_table.shape[1]
    info = plsc.get_sparse_core_info()
    NW = info.num_cores * info.num_subcores
    bpw = B // NW
    mesh = plsc.VectorSubcoreMesh(core_axis_name="c", subcore_axis_name="s")

    pos_idx16 = jnp.full((_L,), input_position, dtype=jnp.int32)

    @functools.partial(
        pl.kernel,
        mesh=mesh,
        compiler_params=pltpu.CompilerParams(
            use_tc_tiling_on_sc=False,
            has_side_effects=pltpu.SideEffectType.PURE,
        ),
        out_type=jax.ShapeDtypeStruct((B, 2 * D), jnp.float32),
        scratch_types=[
            pltpu.VMEM((bpw,), jnp.int32),
            pltpu.VMEM((bpw, D), jnp.float32),
            pltpu.VMEM((_L,), jnp.int32),
            pltpu.VMEM((_L, D), jnp.float32),
            pltpu.VMEM((bpw, 2 * D), jnp.float32),
            pltpu.SemaphoreType.DMA,
            pltpu.SemaphoreType.DMA,
        ],
    )
    def sc_kernel(inp_hbm, pidx_hbm, table_hbm, ptable_hbm, out_hbm,
                  idx_v, emb_v, pidx_v, pos16_v, full_v, sem_e, sem_p):
        wid = lax.axis_index("s") * info.num_cores + lax.axis_index("c")
        base = wid * bpw
        pltpu.sync_copy(inp_hbm.at[pl.ds(base, bpw)], idx_v)
        emb_dma = pltpu.async_copy(table_hbm.at[idx_v], emb_v, sem_e)

        pltpu.sync_copy(pidx_hbm, pidx_v)
        pltpu.async_copy(ptable_hbm.at[pidx_v], pos16_v, sem_p).wait()
        pvecs = [pos16_v[0, pl.ds(k * _L, _L)] for k in range(D // _L)]

        def fill_pos(r, carry):
            for k in range(D // _L):
                full_v[r, pl.ds(D + k * _L, _L)] = pvecs[k]
            return carry

        lax.fori_loop(0, bpw, fill_pos, 0)

        emb_dma.wait()

        def copy_emb(r, carry):
            for k in range(D // _L):
                full_v[r, pl.ds(k * _L, _L)] = emb_v[r, pl.ds(k * _L, _L)]
            return carry

        lax.fori_loop(0, bpw, copy_emb, 0)

        pltpu.sync_copy(full_v, out_hbm.at[pl.ds(base, bpw)])

    return sc_kernel(input, pos_idx16, input_table, pos_table)


# speedup vs baseline: 1.0011x; 1.0011x over previous
import functools

import jax
import jax.numpy as jnp
from jax import lax
from jax.experimental import pallas as pl
from jax.experimental.pallas import tpu as pltpu
from jax.experimental.pallas import tpu_sc as plsc

_L = 16


def kernel(input, input_position, input_table, pos_table):
    B = input.shape[0]
    D = input_table.shape[1]
    info = plsc.get_sparse_core_info()
    NW = info.num_cores * info.num_subcores
    bpw = B // NW
    mesh = plsc.VectorSubcoreMesh(core_axis_name="c", subcore_axis_name="s")

    pos_idx16 = jnp.full((_L,), input_position, dtype=jnp.int32)

    @functools.partial(
        pl.kernel,
        mesh=mesh,
        compiler_params=pltpu.CompilerParams(
            use_tc_tiling_on_sc=False,
            has_side_effects=pltpu.SideEffectType.PURE,
        ),
        out_type=jax.ShapeDtypeStruct((B, 2 * D), jnp.float32),
        scratch_types=[
            pltpu.VMEM((bpw,), jnp.int32),
            pltpu.VMEM((bpw, D), jnp.float32),
            pltpu.VMEM((_L,), jnp.int32),
            pltpu.VMEM((_L, D), jnp.float32),
            pltpu.VMEM((bpw, 2 * D), jnp.float32),
            pltpu.SemaphoreType.DMA,
            pltpu.SemaphoreType.DMA,
        ],
    )
    def sc_kernel(inp_hbm, pidx_hbm, table_hbm, ptable_hbm, out_hbm,
                  idx_v, emb_v, pidx_v, pos16_v, full_v, sem_e, sem_p):
        wid = lax.axis_index("s") * info.num_cores + lax.axis_index("c")
        base = wid * bpw
        pltpu.sync_copy(inp_hbm.at[pl.ds(base, bpw)], idx_v)
        emb_dma = pltpu.async_copy(table_hbm.at[idx_v], emb_v, sem_e)

        pltpu.sync_copy(pidx_hbm, pidx_v)
        pltpu.async_copy(ptable_hbm.at[pidx_v], pos16_v, sem_p).wait()
        pvecs = [pos16_v[0, pl.ds(k * _L, _L)] for k in range(D // _L)]

        def fill_pos(r, carry):
            for k in range(D // _L):
                full_v[r, pl.ds(D + k * _L, _L)] = pvecs[k]
            return carry

        lax.fori_loop(0, bpw, fill_pos, 0)

        emb_dma.wait()

        def copy_emb(r, carry):
            for k in range(D // _L):
                full_v[r, pl.ds(k * _L, _L)] = emb_v[r, pl.ds(k * _L, _L)]
            return carry

        lax.fori_loop(0, bpw, copy_emb, 0)

        pltpu.sync_copy(full_v, out_hbm.at[pl.ds(base, bpw)])

    table_lin = jax.lax.optimization_barrier(input_table.reshape(-1))
    table2d = table_lin.reshape(input_table.shape)
    return sc_kernel(input, pos_idx16, table2d, pos_table)


# speedup vs baseline: 1.1268x; 1.1256x over previous
import functools

import jax
import jax.numpy as jnp
from jax import lax
from jax.experimental import pallas as pl
from jax.experimental.pallas import tpu as pltpu
from jax.experimental.pallas import tpu_sc as plsc

_L = 16


def kernel(input, input_position, input_table, pos_table):
    B = input.shape[0]
    D = input_table.shape[1]
    info = plsc.get_sparse_core_info()
    NW = info.num_cores * info.num_subcores
    bpw = B // NW
    mesh = plsc.VectorSubcoreMesh(core_axis_name="c", subcore_axis_name="s")

    table128 = jnp.pad(input_table, ((0, 0), (0, D)))
    pos_idx16 = jnp.full((_L,), input_position, dtype=jnp.int32)

    @functools.partial(
        pl.kernel,
        mesh=mesh,
        compiler_params=pltpu.CompilerParams(use_tc_tiling_on_sc=False),
        out_type=jax.ShapeDtypeStruct((B, 2 * D), jnp.float32),
        scratch_types=[
            pltpu.VMEM((bpw,), jnp.int32),
            pltpu.VMEM((bpw, 2 * D), jnp.float32),
            pltpu.VMEM((_L,), jnp.int32),
            pltpu.VMEM((_L, D), jnp.float32),
            pltpu.SemaphoreType.DMA,
            pltpu.SemaphoreType.DMA,
        ],
    )
    def sc_kernel(inp_hbm, pidx_hbm, table_hbm, ptable_hbm, out_hbm,
                  idx_v, stripe, pidx_v, pos16_v, sem_e, sem_p):
        wid = lax.axis_index("s") * info.num_cores + lax.axis_index("c")
        base = wid * bpw
        pltpu.sync_copy(inp_hbm.at[pl.ds(base, bpw)], idx_v)
        emb_dma = pltpu.async_copy(table_hbm.at[idx_v], stripe, sem_e)

        pltpu.sync_copy(pidx_hbm, pidx_v)
        pltpu.async_copy(ptable_hbm.at[pidx_v], pos16_v, sem_p).wait()
        pvecs = [pos16_v[0, pl.ds(k * _L, _L)] for k in range(D // _L)]

        emb_dma.wait()

        def fill_pos(r, carry):
            for k in range(D // _L):
                stripe[r, pl.ds(D + k * _L, _L)] = pvecs[k]
            return carry

        lax.fori_loop(0, bpw, fill_pos, 0)

        pltpu.sync_copy(stripe, out_hbm.at[pl.ds(base, bpw)])

    return sc_kernel(input, pos_idx16, table128, pos_table)
